# Initial kernel scaffold; baseline (speedup 1.0000x reference)
import jax
import jax.numpy as jnp
from jax.experimental import pallas as pl


def kernel(trg_token_ids_batch):
    raise NotImplementedError("write your pallas kernel here")


# speedup vs baseline: 8.3884x; 8.3884x over previous
import functools

import jax
import jax.numpy as jnp
from jax.experimental import pallas as pl

_SMOOTHING = 0.1
_CONFIDENCE = 1.0 - _SMOOTHING
_PAD = 0
_VOCAB = 32000
_FILL = _SMOOTHING / (_VOCAB - 2)

_ROWS_PER_BLOCK = 128


def _fill_body(tok_ref, out_ref):
    r, v = out_ref.shape
    tok = tok_ref[...]
    col = jax.lax.broadcasted_iota(jnp.int32, (r, v), 1)
    res = jnp.where(col == tok, jnp.float32(_CONFIDENCE), jnp.float32(_FILL))
    res = jnp.where(col == _PAD, jnp.float32(0.0), res)
    out_ref[...] = jnp.where(tok == _PAD, jnp.float32(0.0), res)


@jax.jit
def kernel(trg_token_ids_batch):
    batch = trg_token_ids_batch.shape[0]
    tok = trg_token_ids_batch.astype(jnp.int32)
    grid = (batch // _ROWS_PER_BLOCK,)
    return pl.pallas_call(
        _fill_body,
        grid=grid,
        in_specs=[pl.BlockSpec((_ROWS_PER_BLOCK, 1), lambda i: (i, 0))],
        out_specs=pl.BlockSpec((_ROWS_PER_BLOCK, _VOCAB), lambda i: (i, 0)),
        out_shape=jax.ShapeDtypeStruct((batch, _VOCAB), jnp.float32),
    )(tok)
